# baseline (speedup 1.0000x reference)
import functools

import jax
import jax.numpy as jnp
from jax import lax
from jax.experimental import pallas as pl
from jax.experimental.pallas import tpu as pltpu
from jax.experimental.pallas import tpu_sc as plsc

Q = 1024
D = 384
N = 100000
KB = 2048
NB = 49
NPAD = KB * NB
NBLK = NB * 16
TOPK = 10
NEG = float("-inf")

NW = 32
NG = 4
QG = Q // NG
QPW = QG // NW


def _tc_body(q_ref, k_ref, bias_ref, sc_ref, bm_ref):
    q = q_ref[...]
    qn = q / jnp.sqrt(jnp.sum(q * q, axis=1, keepdims=True))
    kb = k_ref[...]
    kn = kb / jnp.sqrt(jnp.maximum(
        jnp.sum(kb * kb, axis=1, keepdims=True), 1e-30))
    s = lax.dot_general(qn, kn, (((1,), (1,)), ((), ())),
                        preferred_element_type=jnp.float32)
    s = s + bias_ref[0]
    bms = []
    for j in range(16):
        sl = s[:, j * 128:(j + 1) * 128]
        sc_ref[0, j] = sl
        bms.append(jnp.max(sl, axis=1, keepdims=True))
    bm_ref[0] = jnp.concatenate(bms, axis=1)


def _sc_body(bm_hbm, tab_hbm, vals_hbm, idx_hbm,
             bm_v, blk_v, row_v, rows_v, ov_v, oi_v, sem):
    c = lax.axis_index("c")
    s = lax.axis_index("s")
    wid = s * 2 + c
    iota = lax.broadcasted_iota(jnp.int32, (16,), 0)
    neg = jnp.full((16,), NEG, jnp.float32)
    zero_i = jnp.zeros((16,), jnp.int32)

    def one_query(i, carry):
        q = wid * QPW + i
        pltpu.sync_copy(bm_hbm.at[q], bm_v)
        run_k, run_i = neg, zero_i
        for j in range(NBLK // 16):
            ck = bm_v[pl.ds(j * 16, 16)]
            ci = j * 16 + iota
            ck, ci = plsc.sort_key_val(ck, ci, descending=True)
            m = ck > run_k
            nk = jnp.where(m, ck, run_k)
            ni = jnp.where(m, ci, run_i)
            run_k, run_i = plsc.sort_key_val(nk, ni, descending=False)
        blk_desc = lax.rev(run_i, (0,))
        blk_v[...] = blk_desc
        row_v[...] = blk_desc * QG + q
        pltpu.async_copy(tab_hbm.at[row_v], rows_v, sem).wait()
        r2k, r2i = neg, zero_i
        for r in range(TOPK):
            for cb in range(8):
                ck = rows_v[r, pl.ds(cb * 16, 16)]
                gl = r * 128 + cb * 16 + iota
                ck, gl = plsc.sort_key_val(ck, gl, descending=True)
                m = ck > r2k
                nk = jnp.where(m, ck, r2k)
                ni = jnp.where(m, gl, r2i)
                r2k, r2i = plsc.sort_key_val(nk, ni, descending=False)
        rk_desc = lax.rev(r2k, (0,))
        rl_desc = lax.rev(r2i, (0,))
        rvec = lax.shift_right_logical(rl_desc, 7)
        off = jnp.bitwise_and(rl_desc, 127)
        blk_lane = plsc.load_gather(blk_v, [rvec])
        ov_v[...] = rk_desc
        oi_v[...] = blk_lane * 128 + off
        pltpu.sync_copy(ov_v, vals_hbm.at[q])
        pltpu.sync_copy(oi_v, idx_hbm.at[q])
        return carry

    lax.fori_loop(0, QPW, one_query, 0)


_sc_call = pl.kernel(
    _sc_body,
    out_type=[
        jax.ShapeDtypeStruct((QG, 16), jnp.float32),
        jax.ShapeDtypeStruct((QG, 16), jnp.int32),
    ],
    mesh=plsc.VectorSubcoreMesh(core_axis_name="c", subcore_axis_name="s"),
    scratch_types=[
        pltpu.VMEM((NBLK,), jnp.float32),
        pltpu.VMEM((16,), jnp.int32),
        pltpu.VMEM((16,), jnp.int32),
        pltpu.VMEM((16, 128), jnp.float32),
        pltpu.VMEM((16,), jnp.float32),
        pltpu.VMEM((16,), jnp.int32),
        pltpu.SemaphoreType.DMA,
    ],
    compiler_params=pltpu.CompilerParams(needs_layout_passes=False),
)


_tc_call = pl.pallas_call(
    _tc_body,
    grid=(NB,),
    in_specs=[
        pl.BlockSpec((QG, D), lambda b: (0, 0)),
        pl.BlockSpec((KB, D), lambda b: (b, 0)),
        pl.BlockSpec((1, 1, KB), lambda b: (b, 0, 0)),
    ],
    out_specs=[
        pl.BlockSpec((1, 16, QG, 128), lambda b: (b, 0, 0, 0)),
        pl.BlockSpec((1, QG, 16), lambda b: (b, 0, 0)),
    ],
    out_shape=[
        jax.ShapeDtypeStruct((NB, 16, QG, 128), jnp.float32),
        jax.ShapeDtypeStruct((NB, QG, 16), jnp.float32),
    ],
    compiler_params=pltpu.CompilerParams(
        dimension_semantics=("arbitrary",)),
)


def kernel(queries, keys, k):
    keys_p = jnp.pad(keys, ((0, NPAD - N), (0, 0)))
    bias = jnp.where(jnp.arange(NPAD) < N, 0.0, -jnp.inf)
    bias = bias.astype(jnp.float32).reshape(NB, 1, KB)
    vals_g, idx_g = [], []
    for g in range(NG):
        qg = lax.slice_in_dim(queries, g * QG, (g + 1) * QG, axis=0)
        sc, bm = _tc_call(qg, keys_p, bias)
        tab = sc.reshape(NBLK * QG, 128)
        bmt = bm.transpose(1, 0, 2).reshape(QG, NBLK)
        v16, i16 = _sc_call(bmt, tab)
        vals_g.append(v16)
        idx_g.append(i16)
    vals16 = jnp.concatenate(vals_g, axis=0)
    idx16 = jnp.concatenate(idx_g, axis=0)
    top_vals = vals16[:, :TOPK]
    top_idx = idx16[:, :TOPK] + (jnp.asarray(k, jnp.int32) - TOPK)
    return top_vals, top_idx


# speedup vs baseline: 1.4343x; 1.4343x over previous
import functools

import jax
import jax.numpy as jnp
from jax import lax
from jax.experimental import pallas as pl
from jax.experimental.pallas import tpu as pltpu
from jax.experimental.pallas import tpu_sc as plsc

Q = 1024
D = 384
N = 100000
KB = 2048
NB = 49
NPAD = KB * NB
NBLK = NB * 16
TOPK = 10
NEG = float("-inf")

NW = 32
NG = 1
QG = Q // NG
QPW = QG // NW


def _tc_body(q_ref, k_ref, bias_ref, sc_ref, bm_ref):
    q = q_ref[...]
    qn = q / jnp.sqrt(jnp.sum(q * q, axis=1, keepdims=True))
    kb = k_ref[...]
    kn = kb / jnp.sqrt(jnp.maximum(
        jnp.sum(kb * kb, axis=1, keepdims=True), 1e-30))
    s = lax.dot_general(qn, kn, (((1,), (1,)), ((), ())),
                        preferred_element_type=jnp.float32)
    s = s + bias_ref[0]
    bms = []
    for j in range(16):
        sl = s[:, j * 128:(j + 1) * 128]
        sc_ref[0, j] = sl
        bms.append(jnp.max(sl, axis=1, keepdims=True))
    bm_ref[0] = jnp.concatenate(bms, axis=1)


def _sc_body(bm_hbm, tab_hbm, vals_hbm, idx_hbm,
             bm_v, blk_v, row_v, rows_v, ov_v, oi_v, sem):
    c = lax.axis_index("c")
    s = lax.axis_index("s")
    wid = s * 2 + c
    iota = lax.broadcasted_iota(jnp.int32, (16,), 0)
    neg = jnp.full((16,), NEG, jnp.float32)
    zero_i = jnp.zeros((16,), jnp.int32)

    def one_query(i, carry):
        q = wid * QPW + i
        pltpu.sync_copy(bm_hbm.at[q], bm_v)
        run_k, run_i = neg, zero_i
        for j in range(NBLK // 16):
            ck = bm_v[pl.ds(j * 16, 16)]
            ci = j * 16 + iota
            ck, ci = plsc.sort_key_val(ck, ci, descending=True)
            m = ck > run_k
            nk = jnp.where(m, ck, run_k)
            ni = jnp.where(m, ci, run_i)
            run_k, run_i = plsc.sort_key_val(nk, ni, descending=False)
        blk_desc = lax.rev(run_i, (0,))
        blk_v[...] = blk_desc
        row_v[...] = blk_desc * QG + q
        pltpu.async_copy(tab_hbm.at[row_v], rows_v, sem).wait()
        r2k, r2i = neg, zero_i
        for r in range(TOPK):
            for cb in range(8):
                ck = rows_v[r, pl.ds(cb * 16, 16)]
                gl = r * 128 + cb * 16 + iota
                ck, gl = plsc.sort_key_val(ck, gl, descending=True)
                m = ck > r2k
                nk = jnp.where(m, ck, r2k)
                ni = jnp.where(m, gl, r2i)
                r2k, r2i = plsc.sort_key_val(nk, ni, descending=False)
        rk_desc = lax.rev(r2k, (0,))
        rl_desc = lax.rev(r2i, (0,))
        rvec = lax.shift_right_logical(rl_desc, 7)
        off = jnp.bitwise_and(rl_desc, 127)
        blk_lane = plsc.load_gather(blk_v, [rvec])
        ov_v[...] = rk_desc
        oi_v[...] = blk_lane * 128 + off
        pltpu.sync_copy(ov_v, vals_hbm.at[q])
        pltpu.sync_copy(oi_v, idx_hbm.at[q])
        return carry

    lax.fori_loop(0, QPW, one_query, 0)


_sc_call = pl.kernel(
    _sc_body,
    out_type=[
        jax.ShapeDtypeStruct((QG, 16), jnp.float32),
        jax.ShapeDtypeStruct((QG, 16), jnp.int32),
    ],
    mesh=plsc.VectorSubcoreMesh(core_axis_name="c", subcore_axis_name="s"),
    scratch_types=[
        pltpu.VMEM((NBLK,), jnp.float32),
        pltpu.VMEM((16,), jnp.int32),
        pltpu.VMEM((16,), jnp.int32),
        pltpu.VMEM((16, 128), jnp.float32),
        pltpu.VMEM((16,), jnp.float32),
        pltpu.VMEM((16,), jnp.int32),
        pltpu.SemaphoreType.DMA,
    ],
    compiler_params=pltpu.CompilerParams(needs_layout_passes=False),
)


_tc_call = pl.pallas_call(
    _tc_body,
    grid=(NB,),
    in_specs=[
        pl.BlockSpec((QG, D), lambda b: (0, 0)),
        pl.BlockSpec((KB, D), lambda b: (b, 0)),
        pl.BlockSpec((1, 1, KB), lambda b: (b, 0, 0)),
    ],
    out_specs=[
        pl.BlockSpec((1, 16, QG, 128), lambda b: (b, 0, 0, 0)),
        pl.BlockSpec((1, QG, 16), lambda b: (b, 0, 0)),
    ],
    out_shape=[
        jax.ShapeDtypeStruct((NB, 16, QG, 128), jnp.float32),
        jax.ShapeDtypeStruct((NB, QG, 16), jnp.float32),
    ],
    compiler_params=pltpu.CompilerParams(
        dimension_semantics=("arbitrary",)),
)


def kernel(queries, keys, k):
    keys_p = jnp.pad(keys, ((0, NPAD - N), (0, 0)))
    bias = jnp.where(jnp.arange(NPAD) < N, 0.0, -jnp.inf)
    bias = bias.astype(jnp.float32).reshape(NB, 1, KB)
    vals_g, idx_g = [], []
    for g in range(NG):
        qg = lax.slice_in_dim(queries, g * QG, (g + 1) * QG, axis=0)
        sc, bm = _tc_call(qg, keys_p, bias)
        tab = sc.reshape(NBLK * QG, 128)
        bmt = bm.transpose(1, 0, 2).reshape(QG, NBLK)
        v16, i16 = _sc_call(bmt, tab)
        vals_g.append(v16)
        idx_g.append(i16)
    vals16 = jnp.concatenate(vals_g, axis=0)
    idx16 = jnp.concatenate(idx_g, axis=0)
    top_vals = vals16[:, :TOPK]
    top_idx = idx16[:, :TOPK] + (jnp.asarray(k, jnp.int32) - TOPK)
    return top_vals, top_idx


# speedup vs baseline: 1.4611x; 1.0186x over previous
import functools

import jax
import jax.numpy as jnp
from jax import lax
from jax.experimental import pallas as pl
from jax.experimental.pallas import tpu as pltpu
from jax.experimental.pallas import tpu_sc as plsc

Q = 1024
D = 384
N = 100000
KB = 4096
NB = 25
NPAD = KB * NB
JB = KB // 128
NBLK = NB * JB
TOPK = 10
NEG = float("-inf")

NW = 32
NG = 1
QG = Q // NG
QPW = QG // NW


def _tc_body(q_ref, k_ref, bias_ref, sc_ref, bm_ref):
    q = q_ref[...]
    qn = q / jnp.sqrt(jnp.sum(q * q, axis=1, keepdims=True))
    kb = k_ref[...]
    kn = kb / jnp.sqrt(jnp.maximum(
        jnp.sum(kb * kb, axis=1, keepdims=True), 1e-30))
    s = lax.dot_general(qn, kn, (((1,), (1,)), ((), ())),
                        preferred_element_type=jnp.float32)
    s = s + bias_ref[0]
    bms = []
    for j in range(JB):
        sl = s[:, j * 128:(j + 1) * 128]
        sc_ref[0, j] = sl
        bms.append(jnp.max(sl, axis=1, keepdims=True))
    bm_ref[0] = jnp.concatenate(bms, axis=1)


def _sc_body(bm_hbm, tab_hbm, vals_hbm, idx_hbm,
             bm_v, blk_v, row_v, rows_v, ov_v, oi_v, sem):
    c = lax.axis_index("c")
    s = lax.axis_index("s")
    wid = s * 2 + c
    iota = lax.broadcasted_iota(jnp.int32, (16,), 0)
    neg = jnp.full((16,), NEG, jnp.float32)
    zero_i = jnp.zeros((16,), jnp.int32)

    def one_query(i, carry):
        q = wid * QPW + i
        pltpu.sync_copy(bm_hbm.at[q], bm_v)
        run_k, run_i = neg, zero_i
        for j in range(NBLK // 16):
            ck = bm_v[pl.ds(j * 16, 16)]
            ci = j * 16 + iota
            ck, ci = plsc.sort_key_val(ck, ci, descending=True)
            m = ck > run_k
            nk = jnp.where(m, ck, run_k)
            ni = jnp.where(m, ci, run_i)
            run_k, run_i = plsc.sort_key_val(nk, ni, descending=False)
        blk_desc = lax.rev(run_i, (0,))
        blk_v[...] = blk_desc
        row_v[...] = blk_desc * QG + q
        pltpu.async_copy(tab_hbm.at[row_v], rows_v, sem).wait()
        r2k, r2i = neg, zero_i
        for r in range(TOPK):
            for cb in range(8):
                ck = rows_v[r, pl.ds(cb * 16, 16)]
                gl = r * 128 + cb * 16 + iota
                ck, gl = plsc.sort_key_val(ck, gl, descending=True)
                m = ck > r2k
                nk = jnp.where(m, ck, r2k)
                ni = jnp.where(m, gl, r2i)
                r2k, r2i = plsc.sort_key_val(nk, ni, descending=False)
        rk_desc = lax.rev(r2k, (0,))
        rl_desc = lax.rev(r2i, (0,))
        rvec = lax.shift_right_logical(rl_desc, 7)
        off = jnp.bitwise_and(rl_desc, 127)
        blk_lane = plsc.load_gather(blk_v, [rvec])
        ov_v[...] = rk_desc
        oi_v[...] = blk_lane * 128 + off
        pltpu.sync_copy(ov_v, vals_hbm.at[q])
        pltpu.sync_copy(oi_v, idx_hbm.at[q])
        return carry

    lax.fori_loop(0, QPW, one_query, 0)


_sc_call = pl.kernel(
    _sc_body,
    out_type=[
        jax.ShapeDtypeStruct((QG, 16), jnp.float32),
        jax.ShapeDtypeStruct((QG, 16), jnp.int32),
    ],
    mesh=plsc.VectorSubcoreMesh(core_axis_name="c", subcore_axis_name="s"),
    scratch_types=[
        pltpu.VMEM((NBLK,), jnp.float32),
        pltpu.VMEM((16,), jnp.int32),
        pltpu.VMEM((16,), jnp.int32),
        pltpu.VMEM((16, 128), jnp.float32),
        pltpu.VMEM((16,), jnp.float32),
        pltpu.VMEM((16,), jnp.int32),
        pltpu.SemaphoreType.DMA,
    ],
    compiler_params=pltpu.CompilerParams(needs_layout_passes=False),
)


_tc_call = pl.pallas_call(
    _tc_body,
    grid=(NB,),
    in_specs=[
        pl.BlockSpec((QG, D), lambda b: (0, 0)),
        pl.BlockSpec((KB, D), lambda b: (b, 0)),
        pl.BlockSpec((1, 1, KB), lambda b: (b, 0, 0)),
    ],
    out_specs=[
        pl.BlockSpec((1, JB, QG, 128), lambda b: (b, 0, 0, 0)),
        pl.BlockSpec((1, QG, JB), lambda b: (b, 0, 0)),
    ],
    out_shape=[
        jax.ShapeDtypeStruct((NB, JB, QG, 128), jnp.float32),
        jax.ShapeDtypeStruct((NB, QG, JB), jnp.float32),
    ],
    compiler_params=pltpu.CompilerParams(
        dimension_semantics=("arbitrary",)),
)


def kernel(queries, keys, k):
    keys_p = jnp.pad(keys, ((0, NPAD - N), (0, 0)))
    bias = jnp.where(jnp.arange(NPAD) < N, 0.0, -jnp.inf)
    bias = bias.astype(jnp.float32).reshape(NB, 1, KB)
    vals_g, idx_g = [], []
    for g in range(NG):
        qg = lax.slice_in_dim(queries, g * QG, (g + 1) * QG, axis=0)
        sc, bm = _tc_call(qg, keys_p, bias)
        tab = sc.reshape(NBLK * QG, 128)
        bmt = bm.transpose(1, 0, 2).reshape(QG, NBLK)
        v16, i16 = _sc_call(bmt, tab)
        vals_g.append(v16)
        idx_g.append(i16)
    vals16 = jnp.concatenate(vals_g, axis=0)
    idx16 = jnp.concatenate(idx_g, axis=0)
    top_vals = vals16[:, :TOPK]
    top_idx = idx16[:, :TOPK] + (jnp.asarray(k, jnp.int32) - TOPK)
    return top_vals, top_idx


# speedup vs baseline: 1.4615x; 1.0003x over previous
import functools

import jax
import jax.numpy as jnp
from jax import lax
from jax.experimental import pallas as pl
from jax.experimental.pallas import tpu as pltpu
from jax.experimental.pallas import tpu_sc as plsc

Q = 1024
D = 384
N = 100000
KB = 4096
NB = 25
NPAD = KB * NB
JB = KB // 128
NBLK = NB * JB
TOPK = 10
NEG = float("-inf")

NW = 32
NG = 1
QG = Q // NG
QPW = QG // NW


def _tc_body(q_ref, k_ref, bias_ref, sc_ref, bm_ref):
    q = q_ref[...]
    qn = q / jnp.sqrt(jnp.sum(q * q, axis=1, keepdims=True))
    kb = k_ref[...]
    kn = kb / jnp.sqrt(jnp.maximum(
        jnp.sum(kb * kb, axis=1, keepdims=True), 1e-30))
    s = lax.dot_general(qn, kn, (((1,), (1,)), ((), ())),
                        preferred_element_type=jnp.float32)
    s = s + bias_ref[0]
    bms = []
    for j in range(JB):
        sl = s[:, j * 128:(j + 1) * 128]
        sc_ref[0, j] = sl
        bms.append(jnp.max(sl, axis=1, keepdims=True))
    bm_ref[0] = jnp.concatenate(bms, axis=1)


def _sc_body(bm_hbm, tab_hbm, vals_hbm, idx_hbm,
             bm_v, blk_v, row_v, rows_v, ov_v, oi_v, sem):
    c = lax.axis_index("c")
    s = lax.axis_index("s")
    wid = s * 2 + c
    iota = lax.broadcasted_iota(jnp.int32, (16,), 0)
    neg = jnp.full((16,), NEG, jnp.float32)
    zero_i = jnp.zeros((16,), jnp.int32)

    def one_query(i, carry):
        q = wid * QPW + i
        pltpu.sync_copy(bm_hbm.at[q], bm_v)
        run_k, run_i = neg, zero_i
        for j in range(NBLK // 16):
            ck = bm_v[pl.ds(j * 16, 16)]
            ci = j * 16 + iota
            ck, ci = plsc.sort_key_val(ck, ci, descending=True)
            m = ck > run_k
            nk = jnp.where(m, ck, run_k)
            ni = jnp.where(m, ci, run_i)
            run_k, run_i = plsc.sort_key_val(nk, ni, descending=False)
        blk_desc = lax.rev(run_i, (0,))
        blk_v[...] = blk_desc
        row_v[...] = blk_desc * QG + q
        pltpu.async_copy(tab_hbm.at[row_v], rows_v, sem).wait()
        r2k, r2i = neg, zero_i
        for r in range(TOPK):
            for cb in range(8):
                ck = rows_v[r, pl.ds(cb * 16, 16)]
                gl = r * 128 + cb * 16 + iota
                ck, gl = plsc.sort_key_val(ck, gl, descending=True)
                m = ck > r2k
                nk = jnp.where(m, ck, r2k)
                ni = jnp.where(m, gl, r2i)
                r2k, r2i = plsc.sort_key_val(nk, ni, descending=False)
        rk_desc = lax.rev(r2k, (0,))
        rl_desc = lax.rev(r2i, (0,))
        rvec = lax.shift_right_logical(rl_desc, 7)
        off = jnp.bitwise_and(rl_desc, 127)
        blk_lane = plsc.load_gather(blk_v, [rvec])
        ov_v[...] = rk_desc
        oi_v[...] = blk_lane * 128 + off
        pltpu.sync_copy(ov_v, vals_hbm.at[q])
        pltpu.sync_copy(oi_v, idx_hbm.at[q])
        return carry

    lax.fori_loop(0, QPW, one_query, 0)


_sc_call = pl.kernel(
    _sc_body,
    out_type=[
        jax.ShapeDtypeStruct((QG, 16), jnp.float32),
        jax.ShapeDtypeStruct((QG, 16), jnp.int32),
    ],
    mesh=plsc.VectorSubcoreMesh(core_axis_name="c", subcore_axis_name="s"),
    scratch_types=[
        pltpu.VMEM((NBLK,), jnp.float32),
        pltpu.VMEM((16,), jnp.int32),
        pltpu.VMEM((16,), jnp.int32),
        pltpu.VMEM((16, 128), jnp.float32),
        pltpu.VMEM((16,), jnp.float32),
        pltpu.VMEM((16,), jnp.int32),
        pltpu.SemaphoreType.DMA,
    ],
    compiler_params=pltpu.CompilerParams(needs_layout_passes=False),
)


_tc_call = pl.pallas_call(
    _tc_body,
    grid=(NB,),
    in_specs=[
        pl.BlockSpec((QG, D), lambda b: (0, 0)),
        pl.BlockSpec((KB, D), lambda b: (b, 0)),
        pl.BlockSpec((1, 1, KB), lambda b: (b, 0, 0)),
    ],
    out_specs=[
        pl.BlockSpec((1, JB, QG, 128), lambda b: (b, 0, 0, 0)),
        pl.BlockSpec((1, QG, JB), lambda b: (b, 0, 0)),
    ],
    out_shape=[
        jax.ShapeDtypeStruct((NB, JB, QG, 128), jnp.float32),
        jax.ShapeDtypeStruct((NB, QG, JB), jnp.float32),
    ],
    compiler_params=pltpu.CompilerParams(
        dimension_semantics=("parallel",)),
)


def kernel(queries, keys, k):
    keys_p = jnp.pad(keys, ((0, NPAD - N), (0, 0)))
    bias = jnp.where(jnp.arange(NPAD) < N, 0.0, -jnp.inf)
    bias = bias.astype(jnp.float32).reshape(NB, 1, KB)
    vals_g, idx_g = [], []
    for g in range(NG):
        qg = lax.slice_in_dim(queries, g * QG, (g + 1) * QG, axis=0)
        sc, bm = _tc_call(qg, keys_p, bias)
        tab = sc.reshape(NBLK * QG, 128)
        bmt = bm.transpose(1, 0, 2).reshape(QG, NBLK)
        v16, i16 = _sc_call(bmt, tab)
        vals_g.append(v16)
        idx_g.append(i16)
    vals16 = jnp.concatenate(vals_g, axis=0)
    idx16 = jnp.concatenate(idx_g, axis=0)
    top_vals = vals16[:, :TOPK]
    top_idx = idx16[:, :TOPK] + (jnp.asarray(k, jnp.int32) - TOPK)
    return top_vals, top_idx


# speedup vs baseline: 1.5256x; 1.0439x over previous
import functools

import jax
import jax.numpy as jnp
from jax import lax
from jax.experimental import pallas as pl
from jax.experimental.pallas import tpu as pltpu
from jax.experimental.pallas import tpu_sc as plsc

Q = 1024
D = 384
N = 100000
KB = 4096
NB = 25
NPAD = KB * NB
JB = KB // 128
NBLK = NB * JB
TOPK = 10
NEG = float("-inf")

NW = 32
NG = 1
QG = Q // NG
QPW = QG // NW


def _tc_body(q_ref, k_ref, bias_ref, sc_ref, bm_ref):
    q = q_ref[...]
    qn = q / jnp.sqrt(jnp.sum(q * q, axis=1, keepdims=True))
    kb = k_ref[...]
    kn = kb / jnp.sqrt(jnp.maximum(
        jnp.sum(kb * kb, axis=1, keepdims=True), 1e-30))
    s = lax.dot_general(qn, kn, (((1,), (1,)), ((), ())),
                        preferred_element_type=jnp.float32)
    s = s + bias_ref[0]
    bms = []
    for j in range(JB):
        sl = s[:, j * 128:(j + 1) * 128]
        sc_ref[0, j] = sl
        bms.append(jnp.max(sl, axis=1, keepdims=True))
    bm_ref[0] = jnp.concatenate(bms, axis=1)


def _sc_body(bm_hbm, tab_hbm, vals_hbm, idx_hbm,
             bm_v, blk_v0, row_v0, rows_v0, blk_v1, row_v1, rows_v1,
             ov_v, oi_v, sem0, sem1):
    c = lax.axis_index("c")
    s = lax.axis_index("s")
    wid = s * 2 + c
    iota = lax.broadcasted_iota(jnp.int32, (16,), 0)
    neg = jnp.full((16,), NEG, jnp.float32)
    zero_i = jnp.zeros((16,), jnp.int32)

    def select_and_issue(q, blk_v, row_v, rows_v, sem):
        pltpu.sync_copy(bm_hbm.at[q], bm_v)
        run_k, run_i = neg, zero_i
        for j in range(NBLK // 16):
            ck = bm_v[pl.ds(j * 16, 16)]
            ci = j * 16 + iota
            ck, ci = plsc.sort_key_val(ck, ci, descending=True)
            m = ck > run_k
            nk = jnp.where(m, ck, run_k)
            ni = jnp.where(m, ci, run_i)
            run_k, run_i = plsc.sort_key_val(nk, ni, descending=False)
        blk_desc = lax.rev(run_i, (0,))
        blk_v[...] = blk_desc
        row_v[...] = blk_desc * QG + q
        pltpu.async_copy(tab_hbm.at[row_v], rows_v, sem)

    def finish(q, blk_v, row_v, rows_v, sem):
        pltpu.make_async_copy(tab_hbm.at[row_v], rows_v, sem).wait()
        r2k, r2i = neg, zero_i
        for r in range(TOPK):
            for cb in range(8):
                ck = rows_v[r, pl.ds(cb * 16, 16)]
                gl = r * 128 + cb * 16 + iota
                ck, gl = plsc.sort_key_val(ck, gl, descending=True)
                m = ck > r2k
                nk = jnp.where(m, ck, r2k)
                ni = jnp.where(m, gl, r2i)
                r2k, r2i = plsc.sort_key_val(nk, ni, descending=False)
        rk_desc = lax.rev(r2k, (0,))
        rl_desc = lax.rev(r2i, (0,))
        rvec = lax.shift_right_logical(rl_desc, 7)
        off = jnp.bitwise_and(rl_desc, 127)
        blk_lane = plsc.load_gather(blk_v, [rvec])
        ov_v[...] = rk_desc
        oi_v[...] = blk_lane * 128 + off
        pltpu.sync_copy(ov_v, vals_hbm.at[q])
        pltpu.sync_copy(oi_v, idx_hbm.at[q])

    def step(i, carry):
        even = lax.rem(i, 2) == 0

        @pl.when(i < QPW)
        def _sel():
            q = wid * QPW + i

            @pl.when(even)
            def _():
                select_and_issue(q, blk_v0, row_v0, rows_v0, sem0)

            @pl.when(jnp.logical_not(even))
            def _():
                select_and_issue(q, blk_v1, row_v1, rows_v1, sem1)

        @pl.when(i > 0)
        def _fin():
            qp = wid * QPW + i - 1

            @pl.when(jnp.logical_not(even))
            def _():
                finish(qp, blk_v0, row_v0, rows_v0, sem0)

            @pl.when(even)
            def _():
                finish(qp, blk_v1, row_v1, rows_v1, sem1)

        return carry

    lax.fori_loop(0, QPW + 1, step, 0)


_sc_call = pl.kernel(
    _sc_body,
    out_type=[
        jax.ShapeDtypeStruct((QG, 16), jnp.float32),
        jax.ShapeDtypeStruct((QG, 16), jnp.int32),
    ],
    mesh=plsc.VectorSubcoreMesh(core_axis_name="c", subcore_axis_name="s"),
    scratch_types=[
        pltpu.VMEM((NBLK,), jnp.float32),
        pltpu.VMEM((16,), jnp.int32),
        pltpu.VMEM((16,), jnp.int32),
        pltpu.VMEM((16, 128), jnp.float32),
        pltpu.VMEM((16,), jnp.int32),
        pltpu.VMEM((16,), jnp.int32),
        pltpu.VMEM((16, 128), jnp.float32),
        pltpu.VMEM((16,), jnp.float32),
        pltpu.VMEM((16,), jnp.int32),
        pltpu.SemaphoreType.DMA,
        pltpu.SemaphoreType.DMA,
    ],
    compiler_params=pltpu.CompilerParams(needs_layout_passes=False),
)


_tc_call = pl.pallas_call(
    _tc_body,
    grid=(NB,),
    in_specs=[
        pl.BlockSpec((QG, D), lambda b: (0, 0)),
        pl.BlockSpec((KB, D), lambda b: (b, 0)),
        pl.BlockSpec((1, 1, KB), lambda b: (b, 0, 0)),
    ],
    out_specs=[
        pl.BlockSpec((1, JB, QG, 128), lambda b: (b, 0, 0, 0)),
        pl.BlockSpec((1, QG, JB), lambda b: (b, 0, 0)),
    ],
    out_shape=[
        jax.ShapeDtypeStruct((NB, JB, QG, 128), jnp.float32),
        jax.ShapeDtypeStruct((NB, QG, JB), jnp.float32),
    ],
    compiler_params=pltpu.CompilerParams(
        dimension_semantics=("parallel",)),
)


def kernel(queries, keys, k):
    keys_p = jnp.pad(keys, ((0, NPAD - N), (0, 0)))
    bias = jnp.where(jnp.arange(NPAD) < N, 0.0, -jnp.inf)
    bias = bias.astype(jnp.float32).reshape(NB, 1, KB)
    vals_g, idx_g = [], []
    for g in range(NG):
        qg = lax.slice_in_dim(queries, g * QG, (g + 1) * QG, axis=0)
        sc, bm = _tc_call(qg, keys_p, bias)
        tab = sc.reshape(NBLK * QG, 128)
        bmt = bm.transpose(1, 0, 2).reshape(QG, NBLK)
        v16, i16 = _sc_call(bmt, tab)
        vals_g.append(v16)
        idx_g.append(i16)
    vals16 = jnp.concatenate(vals_g, axis=0)
    idx16 = jnp.concatenate(idx_g, axis=0)
    top_vals = vals16[:, :TOPK]
    top_idx = idx16[:, :TOPK] + (jnp.asarray(k, jnp.int32) - TOPK)
    return top_vals, top_idx
